# baseline (speedup 1.0000x reference)
import jax
import jax.numpy as jnp
from jax import lax
from jax.experimental import pallas as pl
from jax.experimental.pallas import tpu as pltpu
from jax.experimental.pallas import tpu_sc as plsc

B = 4096
H = 50
D = 64
L = 16
NW = 32
NROW = 1000000
NPAIR = NROW // 2
CPAIR = 128
CROW = 2 * CPAIR
GPW = 64
G2 = 2 * H

_OFFS = (0, 16, 32, 48, 64, 80, 84)


def _lane(jp):
    return (jp // 16, jp % 16) if jp < 96 else (6, jp - 84)


def _bcast(reg, lane):
    return reg.at[jnp.full((L,), lane, jnp.int32)].get(mode="promise_in_bounds")


def _wid():
    return lax.axis_index("s") * 2 + lax.axis_index("c")


def _pack_body(src_hbm, out_hbm, in0, in1, ov0, ov1, si0, si1, so0, so1):
    wid = _wid()
    p_lo = (wid * NPAIR // NW) // 8 * 8
    p_hi = ((wid + 1) * NPAIR // NW) // 8 * 8
    nc = (p_hi - p_lo + CPAIR - 1) // CPAIR

    ins = (in0, in1)
    ovs = (ov0, ov1)
    sis = (si0, si1)
    sos = (so0, so1)

    def pstart(i):
        return jnp.minimum(p_lo + i * CPAIR, p_hi - CPAIR)

    for k in range(2):
        pltpu.async_copy(src_hbm.at[pl.ds(2 * pstart(k), CROW)], ins[k],
                         sis[k])

    def step(i, carry):
        k0 = i * 2
        for k in range(2):
            ic = k0 + k
            ps = pstart(ic)
            in_v, ov, si, so = ins[k], ovs[k], sis[k], sos[k]

            @pl.when(ic < nc)
            def _():
                pltpu.make_async_copy(src_hbm.at[pl.ds(2 * ps, CROW)], in_v,
                                      si).wait()

                @pl.when(ic >= 2)
                def _():
                    pltpu.make_async_copy(ov, out_hbm.at[pl.ds(ps, CPAIR)],
                                          so).wait()

                def repack(p, c):
                    r = 2 * p
                    for d in range(D // L):
                        ov[p, pl.ds(L * d, L)] = in_v[r, pl.ds(L * d, L)]
                        ov[p, pl.ds(D + L * d, L)] = in_v[r + 1,
                                                          pl.ds(L * d, L)]
                    return c

                lax.fori_loop(0, CPAIR, repack, 0)
                pltpu.async_copy(ov, out_hbm.at[pl.ds(ps, CPAIR)], so)

                @pl.when(ic + 2 < nc)
                def _():
                    pltpu.async_copy(
                        src_hbm.at[pl.ds(2 * pstart(ic + 2), CROW)], in_v, si)
        return carry

    lax.fori_loop(0, (nc + 1) // 2, step, 0)
    for k in range(2):
        @pl.when(nc > k)
        def _():
            pltpu.make_async_copy(ovs[k], out_hbm.at[pl.ds(p_lo, CPAIR)],
                                  sos[k]).wait()


def _gather_body(feat_hbm, w_hbm, packed_hbm, out_hbm, idx_v, pidx_v, wv,
                 buf0, buf1, out_v, sem0, sem1):
    wid = _wid()
    gbase = wid * GPW

    pltpu.sync_copy(feat_hbm.at[pl.ds(gbase, GPW)], idx_v)
    pltpu.sync_copy(w_hbm.at[pl.ds(gbase, GPW)], wv)

    def prep(g, c):
        for o in _OFFS:
            pidx_v[g, pl.ds(o, L)] = lax.shift_right_logical(
                idx_v[g, pl.ds(o, L)], 1)
        return c

    lax.fori_loop(0, GPW, prep, 0)

    bufs = (buf0, buf1)
    sems = (sem0, sem1)

    for k in range(2):
        pltpu.async_copy(packed_hbm.at[pidx_v.at[k]], bufs[k], sems[k])

    def step(i, carry):
        g0 = i * 2
        for k in range(2):
            g = g0 + k
            buf, sem = bufs[k], sems[k]
            pltpu.make_async_copy(packed_hbm.at[pidx_v.at[g]], buf,
                                  sem).wait()
            wregs = [wv[g, pl.ds(o, L)] for o in _OFFS]
            bregs = [(idx_v[g, pl.ds(o, L)] & 1).astype(jnp.float32)
                     for o in _OFFS]
            for r in range(2):
                acc = [jnp.zeros((L,), jnp.float32) for _ in range(D // L)]
                for j in range(H):
                    jp = r * H + j
                    ri, lane = _lane(jp)
                    w = _bcast(wregs[ri], lane)
                    w1 = w * _bcast(bregs[ri], lane)
                    w0 = w - w1
                    for d in range(D // L):
                        e0 = buf[jp, pl.ds(L * d, L)]
                        e1 = buf[jp, pl.ds(D + L * d, L)]
                        acc[d] = acc[d] + e0 * w0 + e1 * w1
                b = 2 * g + r
                for d in range(D // L):
                    out_v[b, pl.ds(L * d, L)] = acc[d]
            ng = g + 2

            @pl.when(ng < GPW)
            def _():
                pltpu.async_copy(packed_hbm.at[pidx_v.at[ng]], buf, sem)
        return carry

    lax.fori_loop(0, GPW // 2, step, 0)

    pltpu.sync_copy(out_v, out_hbm.at[pl.ds(wid * (B // NW), B // NW)])


@jax.jit
def kernel(features, weights, table):
    mesh = plsc.VectorSubcoreMesh(core_axis_name="c", subcore_axis_name="s")
    params = pltpu.CompilerParams(use_tc_tiling_on_sc=True)

    pack = pl.kernel(
        _pack_body,
        out_type=jax.ShapeDtypeStruct((NPAIR, 2 * D), jnp.float32),
        mesh=mesh,
        scratch_types=[
            pltpu.VMEM((CROW, D), jnp.float32),
            pltpu.VMEM((CROW, D), jnp.float32),
            pltpu.VMEM((CPAIR, 2 * D), jnp.float32),
            pltpu.VMEM((CPAIR, 2 * D), jnp.float32),
            pltpu.SemaphoreType.DMA,
            pltpu.SemaphoreType.DMA,
            pltpu.SemaphoreType.DMA,
            pltpu.SemaphoreType.DMA,
        ],
        compiler_params=params,
    )

    gather = pl.kernel(
        _gather_body,
        out_type=jax.ShapeDtypeStruct((B, D), jnp.float32),
        mesh=mesh,
        scratch_types=[
            pltpu.VMEM((GPW, G2), jnp.int32),
            pltpu.VMEM((GPW, G2), jnp.int32),
            pltpu.VMEM((GPW, G2), jnp.float32),
            pltpu.VMEM((G2, 2 * D), jnp.float32),
            pltpu.VMEM((G2, 2 * D), jnp.float32),
            pltpu.VMEM((B // NW, D), jnp.float32),
            pltpu.SemaphoreType.DMA,
            pltpu.SemaphoreType.DMA,
        ],
        compiler_params=params,
    )

    packed = pack(table)
    featR = features.reshape(NW * GPW, G2)
    wR = weights.reshape(NW * GPW, G2)
    return gather(featR, wR, packed)


# speedup vs baseline: 1.3848x; 1.3848x over previous
import jax
import jax.numpy as jnp
from jax import lax
from jax.experimental import pallas as pl
from jax.experimental.pallas import tpu as pltpu
from jax.experimental.pallas import tpu_sc as plsc

B = 4096
H = 50
D = 64
L = 16
NW = 32
GPW = 64
G2 = 2 * H

_OFFS = (0, 16, 32, 48, 64, 80, 84)


def _lane(jp):
    return (jp // 16, jp % 16) if jp < 96 else (6, jp - 84)


def _bcast(reg, lane):
    return reg.at[jnp.full((L,), lane, jnp.int32)].get(mode="promise_in_bounds")


def _body(feat_hbm, w_hbm, table_hbm, out_hbm, idx_v, wv, buf0, buf1, out_v,
          sem0, sem1):
    wid = lax.axis_index("s") * 2 + lax.axis_index("c")
    gbase = wid * GPW

    pltpu.sync_copy(feat_hbm.at[pl.ds(gbase, GPW)], idx_v)
    pltpu.sync_copy(w_hbm.at[pl.ds(gbase, GPW)], wv)

    bufs = (buf0, buf1)
    sems = (sem0, sem1)

    for k in range(2):
        pltpu.async_copy(table_hbm.at[idx_v.at[k]], bufs[k], sems[k])

    def step(i, carry):
        g0 = i * 2
        for k in range(2):
            g = g0 + k
            buf, sem = bufs[k], sems[k]
            pltpu.make_async_copy(table_hbm.at[idx_v.at[g]], buf, sem).wait()
            wregs = [wv[g, pl.ds(o, L)] for o in _OFFS]
            for r in range(2):
                acc = [jnp.zeros((L,), jnp.float32) for _ in range(D // L)]
                for j in range(H):
                    jp = r * H + j
                    ri, lane = _lane(jp)
                    w = _bcast(wregs[ri], lane)
                    for d in range(D // L):
                        acc[d] = acc[d] + buf[jp, pl.ds(L * d, L)] * w
                b = 2 * g + r
                for d in range(D // L):
                    out_v[b, pl.ds(L * d, L)] = acc[d]
            ng = g + 2

            @pl.when(ng < GPW)
            def _():
                pltpu.async_copy(table_hbm.at[idx_v.at[ng]], buf, sem)
        return carry

    lax.fori_loop(0, GPW // 2, step, 0)

    pltpu.sync_copy(out_v, out_hbm.at[pl.ds(wid * (B // NW), B // NW)])


@jax.jit
def kernel(features, weights, table):
    mesh = plsc.VectorSubcoreMesh(core_axis_name="c", subcore_axis_name="s")
    run = pl.kernel(
        _body,
        out_type=jax.ShapeDtypeStruct((B, D), jnp.float32),
        mesh=mesh,
        scratch_types=[
            pltpu.VMEM((GPW, G2), jnp.int32),
            pltpu.VMEM((GPW, G2), jnp.float32),
            pltpu.VMEM((G2, D), jnp.float32),
            pltpu.VMEM((G2, D), jnp.float32),
            pltpu.VMEM((B // NW, D), jnp.float32),
            pltpu.SemaphoreType.DMA,
            pltpu.SemaphoreType.DMA,
        ],
        compiler_params=pltpu.CompilerParams(use_tc_tiling_on_sc=False),
    )
    return run(features.reshape(NW * GPW, G2), weights.reshape(NW * GPW, G2),
               table)


# speedup vs baseline: 1.3849x; 1.0001x over previous
import jax
import jax.numpy as jnp
from jax import lax
from jax.experimental import pallas as pl
from jax.experimental.pallas import tpu as pltpu
from jax.experimental.pallas import tpu_sc as plsc

B = 4096
H = 50
D = 64
L = 16
NW = 32
GPW = 64
G2 = 2 * H

_OFFS = (0, 16, 32, 48, 64, 80, 84)


def _lane(jp):
    return (jp // 16, jp % 16) if jp < 96 else (6, jp - 84)


def _bcast(reg, lane):
    return reg.at[jnp.full((L,), lane, jnp.int32)].get(mode="promise_in_bounds")


def _body(feat_hbm, w_hbm, table_hbm, out_hbm, idx_v, wv, buf0, buf1, out_v,
          sem0, sem1):
    wid = lax.axis_index("s") * 2 + lax.axis_index("c")
    gbase = wid * GPW

    pltpu.sync_copy(feat_hbm.at[pl.ds(gbase, GPW)], idx_v)
    pltpu.sync_copy(w_hbm.at[pl.ds(gbase, GPW)], wv)

    bufs = (buf0, buf1)
    sems = (sem0, sem1)

    for k in range(2):
        pltpu.async_copy(table_hbm.at[idx_v.at[k]], bufs[k], sems[k])

    def step(i, carry):
        g0 = i * 2
        for k in range(2):
            g = g0 + k
            buf, sem = bufs[k], sems[k]
            pltpu.make_async_copy(table_hbm.at[idx_v.at[g]], buf, sem).wait()
            wregs = [wv[g, pl.ds(o, L)] for o in _OFFS]
            for r in range(2):
                acc = [jnp.zeros((L,), jnp.float32) for _ in range(D // L)]
                acc2 = [jnp.zeros((L,), jnp.float32) for _ in range(D // L)]
                for j in range(H):
                    jp = r * H + j
                    ri, lane = _lane(jp)
                    w = _bcast(wregs[ri], lane)
                    a = acc if j % 2 == 0 else acc2
                    for d in range(D // L):
                        a[d] = a[d] + buf[jp, pl.ds(L * d, L)] * w
                b = 2 * g + r
                for d in range(D // L):
                    out_v[b, pl.ds(L * d, L)] = acc[d] + acc2[d]
            ng = g + 2

            @pl.when(ng < GPW)
            def _():
                pltpu.async_copy(table_hbm.at[idx_v.at[ng]], buf, sem)
        return carry

    lax.fori_loop(0, GPW // 2, step, 0)

    pltpu.sync_copy(out_v, out_hbm.at[pl.ds(wid * (B // NW), B // NW)])


@jax.jit
def kernel(features, weights, table):
    mesh = plsc.VectorSubcoreMesh(core_axis_name="c", subcore_axis_name="s")
    run = pl.kernel(
        _body,
        out_type=jax.ShapeDtypeStruct((B, D), jnp.float32),
        mesh=mesh,
        scratch_types=[
            pltpu.VMEM((GPW, G2), jnp.int32),
            pltpu.VMEM((GPW, G2), jnp.float32),
            pltpu.VMEM((G2, D), jnp.float32),
            pltpu.VMEM((G2, D), jnp.float32),
            pltpu.VMEM((B // NW, D), jnp.float32),
            pltpu.SemaphoreType.DMA,
            pltpu.SemaphoreType.DMA,
        ],
        compiler_params=pltpu.CompilerParams(use_tc_tiling_on_sc=False),
    )
    return run(features.reshape(NW * GPW, G2), weights.reshape(NW * GPW, G2),
               table)
